# baseline (speedup 1.0000x reference)
import jax
import jax.numpy as jnp
from jax import lax
from jax.experimental import pallas as pl
from jax.experimental.pallas import tpu as pltpu
from jax.experimental.pallas import tpu_sc as plsc

D = 64
CHUNK = 128
NBUF = 4
TW = 15872


def _transpose_kernel(tt_ref, out_ref):
    t = tt_ref[...].T
    out_ref[:, :D] = t[:TW // 2]
    out_ref[:, D:] = t[TW // 2:]


def _row_major_table(table):
    v, d = table.shape
    tt = table.T
    grid = (v + TW - 1) // TW
    return pl.pallas_call(
        _transpose_kernel,
        grid=(grid,),
        in_specs=[pl.BlockSpec((d, TW), lambda i: (0, i))],
        out_specs=pl.BlockSpec((TW // 2, 2 * d), lambda i: (i, 0)),
        out_shape=jax.ShapeDtypeStruct((grid * TW // 2, 2 * d), jnp.float32),
    )(tt)


def _remap(r):
    i = r // TW
    j = r % TW
    return i * TW + 2 * (j % (TW // 2)) + j // (TW // 2)


def _out_transpose_kernel(in_ref, out_ref):
    out_ref[...] = in_ref[...].T


def _to_batch_minor(flat, nb, nt):
    bw = 128
    out = pl.pallas_call(
        _out_transpose_kernel,
        grid=(nb // bw,),
        in_specs=[pl.BlockSpec((bw, nt * D), lambda i: (i, 0))],
        out_specs=pl.BlockSpec((nt * D, bw), lambda i: (0, i)),
        out_shape=jax.ShapeDtypeStruct((nt * D, nb), jnp.float32),
    )(flat.reshape(nb, nt * D))
    return out.reshape(nt, D, nb).transpose(2, 0, 1)


def _gather_body(nch, b_per_w, nc,
                 x_hbm, table_hbm, out_hbm,
                 idx_v, rows, gsems, osems):
    wid = lax.axis_index("s") * nc + lax.axis_index("c")
    base = wid * b_per_w

    pltpu.sync_copy(x_hbm.at[wid], idx_v)

    def gather_start(j, b):
        pltpu.make_async_copy(table_hbm.at[idx_v.at[j]], rows[b], gsems[b]).start()

    def gather_wait(j, b):
        pltpu.make_async_copy(table_hbm.at[idx_v.at[j]], rows[b], gsems[b]).wait()

    def out_copy(j, b):
        dst = out_hbm.at[pl.ds(base + j * CHUNK, CHUNK)]
        cp = pltpu.make_async_copy(rows[b], dst, osems[b])
        cp.start()
        return cp

    for b in range(NBUF):
        gather_start(b, b)

    def step(g, carry):
        for b in range(NBUF):
            j = g * NBUF + b
            gather_wait(j, b)
            out_copy(j, b).wait()

            @pl.when(j + NBUF < nch)
            def _():
                gather_start(j + NBUF, b)
        return carry

    lax.fori_loop(0, nch // NBUF, step, 0)


def kernel(x, table):
    orig_shape = x.shape
    b = 1
    for s in orig_shape:
        b *= s

    info = plsc.get_sparse_core_info()
    nc, ns = info.num_cores, info.num_subcores
    nw = nc * ns
    b_per_w = b // nw
    nch = b_per_w // CHUNK
    assert b == nw * nch * CHUNK and nch % NBUF == 0

    xr = _remap(x.reshape(nw, nch, CHUNK).astype(jnp.int32))
    tp = _row_major_table(table)
    tf = tp.reshape(tp.shape[0] * 2, D)
    mesh = plsc.VectorSubcoreMesh(core_axis_name="c", subcore_axis_name="s")

    scratch = [pltpu.VMEM((nch, CHUNK), jnp.int32)]
    scratch += [pltpu.VMEM((CHUNK, D), jnp.float32) for _ in range(NBUF)]
    scratch += [pltpu.SemaphoreType.DMA for _ in range(2 * NBUF)]

    def body(x_hbm, table_hbm, out_hbm, idx_v, *rest):
        rows = rest[:NBUF]
        gsems = rest[NBUF:2 * NBUF]
        osems = rest[2 * NBUF:]
        _gather_body(nch, b_per_w, nc,
                     x_hbm, table_hbm, out_hbm, idx_v, rows, gsems, osems)

    out = pl.kernel(
        body,
        mesh=mesh,
        out_type=jax.ShapeDtypeStruct((b, D), jnp.float32),
        scratch_types=scratch,
        compiler_params=pltpu.CompilerParams(use_tc_tiling_on_sc=False),
    )(xr, tf)
    return _to_batch_minor(out, orig_shape[0], orig_shape[1])
